# baseline (speedup 1.0000x reference)
import functools

import jax
import jax.numpy as jnp
from jax import lax
from jax.experimental import pallas as pl
from jax.experimental.pallas import tpu as pltpu
from jax.experimental.pallas import tpu_sc as plsc

NUM_CLASSES = 1000
BATCH = 16384

NC = 2
NS = 16
L = 16
NW = NC * NS

ROWS_PER_W = BATCH // NW
HALF = ROWS_PER_W // 2
ZROWS = ROWS_PER_W // NS

_ZCOLS = tuple(range(0, NUM_CLASSES - L, L)) + (NUM_CLASSES - L,)


def _one_hot_body(x_hbm, out_hbm, idx_v, zbuf, ident, drain_v,
                  shared_z, sem_fill, sem_a, sem_b, sem_one):
    cid = lax.axis_index("c")
    sid = lax.axis_index("s")
    wid = sid * NC + cid
    rbase = wid * ROWS_PER_W

    pltpu.sync_copy(x_hbm.at[pl.ds(rbase, ROWS_PER_W)], idx_v)

    zeros16 = jnp.zeros((L,), jnp.float32)
    lane = lax.iota(jnp.int32, L)

    def _zero_row(r, carry):
        for c in _ZCOLS:
            zbuf[r, pl.ds(c, L)] = zeros16
        return carry

    lax.fori_loop(0, ZROWS, _zero_row, 0)
    for o in range(L):
        ident[o, :] = jnp.where(lane == o, 1.0, 0.0).astype(jnp.float32)

    fill_cp = pltpu.async_copy(
        zbuf, shared_z.at[pl.ds(sid * ZROWS, ZROWS)], sem_fill)
    fill_cp.wait()
    plsc.subcore_barrier()

    bulk_a = pltpu.async_copy(
        shared_z.at[pl.ds(0, HALF)],
        out_hbm.at[pl.ds(rbase, HALF)], sem_a)
    bulk_b = pltpu.async_copy(
        shared_z.at[pl.ds(HALF, HALF)],
        out_hbm.at[pl.ds(rbase + HALF, HALF)], sem_b)

    def _ones(base):
        def _one(q, carry):
            v = idx_v[pl.ds(base + q * L, L)]
            for j in range(L):
                c = v[j]
                s = pl.multiple_of(
                    jnp.minimum((c >> 3) << 3, NUM_CLASSES - L), 8)
                pltpu.async_copy(
                    ident.at[c - s],
                    out_hbm.at[rbase + base + q * L + j, pl.ds(s, L)],
                    sem_one)
            return carry
        lax.fori_loop(0, HALF // L, _one, 0)

    bulk_a.wait()
    _ones(0)
    bulk_b.wait()
    _ones(HALF)

    pltpu.make_async_copy(
        x_hbm.at[pl.ds(0, ROWS_PER_W * L)], drain_v, sem_one).wait()


_one_hot_sc = functools.partial(
    pl.kernel,
    out_type=jax.ShapeDtypeStruct((BATCH, NUM_CLASSES), jnp.float32),
    mesh=plsc.VectorSubcoreMesh(core_axis_name="c", subcore_axis_name="s"),
    compiler_params=pltpu.CompilerParams(
        needs_layout_passes=False, use_tc_tiling_on_sc=False),
    scratch_types=[
        pltpu.VMEM((ROWS_PER_W,), jnp.int32),
        pltpu.VMEM((ZROWS, NUM_CLASSES), jnp.float32),
        pltpu.VMEM((L, L), jnp.float32),
        pltpu.VMEM((ROWS_PER_W * L,), jnp.int32),
        pltpu.VMEM_SHARED((ROWS_PER_W, NUM_CLASSES), jnp.float32),
        pltpu.SemaphoreType.DMA,
        pltpu.SemaphoreType.DMA,
        pltpu.SemaphoreType.DMA,
        pltpu.SemaphoreType.DMA,
    ],
)(_one_hot_body)


def kernel(x):
    return _one_hot_sc(jnp.reshape(x, (BATCH,)))
